# baseline (speedup 1.0000x reference)
import functools

import jax
import jax.numpy as jnp
from jax import lax
from jax.experimental import pallas as pl
from jax.experimental.pallas import tpu as pltpu
from jax.experimental.pallas import tpu_sc as plsc

N = 10000
E = 320000
D = 128
DE = 16
H = 128
G = 64

_info = plsc.get_sparse_core_info()
NC = _info.num_cores
NS = _info.num_subcores
NW = NC * NS
EPW = E // NW
CH = 40
NCHUNK = EPW // CH
CHS = 80
NCHS = EPW // CHS
ZR = 40
WTILES = 10
RPS = N // WTILES

_mesh = plsc.VectorSubcoreMesh(core_axis_name="c", subcore_axis_name="s")


def _tables_body(x_ref, wd_ref, ws_ref, b_ref, td_ref, ts_ref):
    xb = x_ref[...].astype(jnp.bfloat16)
    td_ref[...] = (
        jnp.dot(xb, wd_ref[...].astype(jnp.bfloat16),
                preferred_element_type=jnp.float32) + b_ref[...]
    )
    ts_ref[...] = jnp.dot(
        xb, ws_ref[...].astype(jnp.bfloat16),
        preferred_element_type=jnp.float32,
    )


def _tables(x, wd, wsr, bcat):
    bm = 1000
    return pl.pallas_call(
        _tables_body,
        grid=(N // bm,),
        in_specs=[
            pl.BlockSpec((bm, D), lambda i: (i, 0)),
            pl.BlockSpec((D, 2 * D), lambda i: (0, 0)),
            pl.BlockSpec((D, 2 * D), lambda i: (0, 0)),
            pl.BlockSpec((1, 2 * D), lambda i: (0, 0)),
        ],
        out_specs=[
            pl.BlockSpec((bm, 2 * D), lambda i: (i, 0)),
            pl.BlockSpec((bm, 2 * D), lambda i: (i, 0)),
        ],
        out_shape=[
            jax.ShapeDtypeStruct((N, 2 * D), jnp.float32),
            jax.ShapeDtypeStruct((N, 2 * D), jnp.float32),
        ],
    )(x, wd, wsr, bcat)


NBUF = 3
AHEAD = 2
NCMAIN = NCHUNK - (NCHUNK % NBUF)


@functools.partial(
    pl.kernel,
    out_type=jax.ShapeDtypeStruct((E, 2 * D), jnp.float32),
    mesh=_mesh,
    scratch_types=[
        pltpu.VMEM((NCHUNK, CH), jnp.int32),
        pltpu.VMEM((NCHUNK, CH), jnp.int32),
        pltpu.VMEM((NBUF, CH, 2 * D), jnp.float32),
        pltpu.VMEM((NBUF, CH, 2 * D), jnp.float32),
        pltpu.SemaphoreType.DMA((NBUF,)),
        pltpu.SemaphoreType.DMA((NBUF,)),
    ],
)
def _gather_k(td_hbm, ts_hbm, dst_hbm, src_hbm, g_hbm, idxd, idxs,
              bufd, bufs, sg, sw):
    wid = lax.axis_index("s") * NC + lax.axis_index("c")
    pltpu.sync_copy(dst_hbm.at[wid], idxd)
    pltpu.sync_copy(src_hbm.at[wid], idxs)
    base = wid * EPW

    def gathers(c, slot):
        return (
            pltpu.make_async_copy(td_hbm.at[idxd.at[c]], bufd.at[slot],
                                  sg.at[slot]),
            pltpu.make_async_copy(ts_hbm.at[idxs.at[c]], bufs.at[slot],
                                  sg.at[slot]),
        )

    def write(c, slot):
        rows = pl.ds(base + c * CH, CH)
        return pltpu.make_async_copy(bufd.at[slot], g_hbm.at[rows],
                                     sw.at[slot])

    def add_into(slot):
        def add_row(r, _):
            for j in range(2 * D // 16):
                sl = pl.ds(j * 16, 16)
                bufd[slot, r, sl] = bufd[slot, r, sl] + bufs[slot, r, sl]
            return 0

        lax.fori_loop(0, CH, add_row, 0, unroll=2)

    for i in range(AHEAD):
        ga, gb = gathers(i, i)
        ga.start()
        gb.start()

    def outer(cc, _):
        for i in range(NBUF):
            c = cc * NBUF + i
            ga, gb = gathers(c, i)
            ga.wait()
            gb.wait()
            add_into(i)
            write(c, i).start()
            j = (i + AHEAD) % NBUF
            cn = c + AHEAD

            @pl.when(cn >= NBUF)
            def _drain():
                write(cn - NBUF, j).wait()

            @pl.when(cn < NCMAIN)
            def _prefetch():
                na, nb = gathers(cn, j)
                na.start()
                nb.start()
        return 0

    lax.fori_loop(0, NCMAIN // NBUF, outer, 0)
    for k in range(NBUF - AHEAD):
        c = NCMAIN - (NBUF - AHEAD) + k
        write(c, c % NBUF).wait()
    for t in range(NCHUNK - NCMAIN):
        c = NCMAIN + t
        ga, gb = gathers(c, t)
        ga.start()
        gb.start()
        ga.wait()
        gb.wait()
        add_into(t)
        write(c, t).start()
        write(c, t).wait()


def _zz_body(g_ref, ea_ref, we_ref, zz_ref):
    ew = jnp.dot(ea_ref[...].astype(jnp.bfloat16),
                 we_ref[...].astype(jnp.bfloat16),
                 preferred_element_type=jnp.float32)
    zz_ref[...] = g_ref[...] + ew


def _edge_zz(g, ea, we):
    bm = 8000
    return pl.pallas_call(
        _zz_body,
        grid=(E // bm,),
        in_specs=[
            pl.BlockSpec((bm, 2 * D), lambda i: (i, 0)),
            pl.BlockSpec((bm, DE), lambda i: (i, 0)),
            pl.BlockSpec((DE, 2 * D), lambda i: (0, 0)),
        ],
        out_specs=pl.BlockSpec((bm, 2 * D), lambda i: (i, 0)),
        out_shape=jax.ShapeDtypeStruct((E, 2 * D), jnp.float32),
    )(g, ea, we)


@functools.partial(
    pl.kernel,
    out_type=jax.ShapeDtypeStruct((NC, N, D), jnp.float32),
    mesh=_mesh,
    scratch_types=[
        pltpu.VMEM((NCHS, CHS), jnp.int32),
        pltpu.VMEM((CHS, D), jnp.float32),
        pltpu.VMEM((ZR, D), jnp.float32),
        pltpu.VMEM_SHARED((N, D), jnp.float32),
        pltpu.SemaphoreType.DMA,
    ],
)
def _scatter_k(msg_hbm, dst_hbm, aggp_hbm, idxd, mbuf, zbuf, aggsh, sem):
    cid = lax.axis_index("c")
    sid = lax.axis_index("s")
    wid = sid * NC + cid

    def zrow(i, _):
        for j in range(D // 16):
            zbuf[i, pl.ds(j * 16, 16)] = jnp.zeros((16,), jnp.float32)
        return 0

    lax.fori_loop(0, ZR, zrow, 0)

    @pl.when(sid < WTILES)
    def _init():
        for t in range(RPS // ZR):
            pltpu.sync_copy(zbuf, aggsh.at[pl.ds(sid * RPS + t * ZR, ZR)])

    plsc.subcore_barrier()

    pltpu.sync_copy(dst_hbm.at[wid], idxd)

    def chunk(c, _):
        pltpu.sync_copy(msg_hbm.at[pl.ds(wid * EPW + c * CHS, CHS)], mbuf)
        pltpu.sync_copy(mbuf, aggsh.at[idxd.at[c]], add=True)
        return 0

    lax.fori_loop(0, NCHS, chunk, 0)
    plsc.subcore_barrier()

    @pl.when(sid < WTILES)
    def _writeout():
        pltpu.sync_copy(
            aggsh.at[pl.ds(sid * RPS, RPS)],
            aggp_hbm.at[cid, pl.ds(sid * RPS, RPS)],
        )


def _head1_body(h_ref, w1_ref, b1_ref, z_ref):
    z_ref[...] = (
        jnp.dot(h_ref[...].astype(jnp.bfloat16),
                w1_ref[...].astype(jnp.bfloat16),
                preferred_element_type=jnp.float32) + b1_ref[...]
    )


def _head1(h, w1, b1):
    return pl.pallas_call(
        _head1_body,
        out_shape=jax.ShapeDtypeStruct((N, H), jnp.float32),
    )(h, w1, b1)


def _head2_body(h2_ref, b_ref, w2_ref, b2_ref, out_ref):
    oh = (
        b_ref[...] == lax.broadcasted_iota(jnp.int32, (N, G), 1)
    ).astype(jnp.float32)
    pooled = lax.dot_general(
        oh, h2_ref[...], (((0,), (0,)), ((), ())),
        preferred_element_type=jnp.float32,
        precision=lax.Precision.HIGHEST,
    )
    out = jnp.dot(pooled.astype(jnp.bfloat16), w2_ref[...].astype(jnp.bfloat16),
                  preferred_element_type=jnp.float32)
    out_ref[...] = jnp.maximum(out + b2_ref[...], 0.0)


def _head2(h2, batch2d, w2, b2):
    return pl.pallas_call(
        _head2_body,
        out_shape=jax.ShapeDtypeStruct((G, 1), jnp.float32),
    )(h2, batch2d, w2, b2)


def kernel(x, edge_index, edge_attr, batch, Wf, bf, Ws, bs, W1, b1, W2, b2):
    src = edge_index[0]
    dst = edge_index[1]
    wd = jnp.concatenate([Wf[:D], Ws[:D]], axis=1)
    wsr = jnp.concatenate([Wf[D : 2 * D], Ws[D : 2 * D]], axis=1)
    we = jnp.concatenate([Wf[2 * D :], Ws[2 * D :]], axis=1)
    bcat = jnp.concatenate([bf, bs]).reshape(1, 2 * D)
    td, ts = _tables(x, wd, wsr, bcat)
    dst3 = dst.reshape(NW, NCHUNK, CH)
    src3 = src.reshape(NW, NCHUNK, CH)
    g = _gather_k(td, ts, dst3, src3)
    zz = _edge_zz(g, edge_attr, we)
    msg = jax.nn.sigmoid(zz[:, :D]) * jax.nn.softplus(zz[:, D:])
    aggp = _scatter_k(msg, dst.reshape(NW, NCHS, CHS))
    h = jax.nn.sigmoid(x + aggp[0] + aggp[1])
    z2 = _head1(h, W1, b1.reshape(1, H))
    h2 = jax.nn.sigmoid(z2)
    return _head2(h2, batch.reshape(N, 1), W2, b2.reshape(1, 1))


# speedup vs baseline: 1.0013x; 1.0013x over previous
import functools

import jax
import jax.numpy as jnp
from jax import lax
from jax.experimental import pallas as pl
from jax.experimental.pallas import tpu as pltpu
from jax.experimental.pallas import tpu_sc as plsc

N = 10000
E = 320000
D = 128
DE = 16
H = 128
G = 64

_info = plsc.get_sparse_core_info()
NC = _info.num_cores
NS = _info.num_subcores
NW = NC * NS
EPW = E // NW
CH = 40
NCHUNK = EPW // CH
CHS = 80
NCHS = EPW // CHS
ZR = 40
WTILES = 10
RPS = N // WTILES

_mesh = plsc.VectorSubcoreMesh(core_axis_name="c", subcore_axis_name="s")


def _tables_body(x_ref, wd_ref, ws_ref, b_ref, td_ref, ts_ref):
    xb = x_ref[...].astype(jnp.bfloat16)
    td_ref[...] = (
        jnp.dot(xb, wd_ref[...].astype(jnp.bfloat16),
                preferred_element_type=jnp.float32) + b_ref[...]
    )
    ts_ref[...] = jnp.dot(
        xb, ws_ref[...].astype(jnp.bfloat16),
        preferred_element_type=jnp.float32,
    )


def _tables(x, wd, wsr, bcat):
    bm = 1000
    return pl.pallas_call(
        _tables_body,
        grid=(N // bm,),
        in_specs=[
            pl.BlockSpec((bm, D), lambda i: (i, 0)),
            pl.BlockSpec((D, 2 * D), lambda i: (0, 0)),
            pl.BlockSpec((D, 2 * D), lambda i: (0, 0)),
            pl.BlockSpec((1, 2 * D), lambda i: (0, 0)),
        ],
        out_specs=[
            pl.BlockSpec((bm, 2 * D), lambda i: (i, 0)),
            pl.BlockSpec((bm, 2 * D), lambda i: (i, 0)),
        ],
        out_shape=[
            jax.ShapeDtypeStruct((N, 2 * D), jnp.float32),
            jax.ShapeDtypeStruct((N, 2 * D), jnp.float32),
        ],
    )(x, wd, wsr, bcat)


NBUF = 3
AHEAD = 2
NCMAIN = NCHUNK - (NCHUNK % NBUF)


@functools.partial(
    pl.kernel,
    out_type=jax.ShapeDtypeStruct((E, 2 * D), jnp.float32),
    mesh=_mesh,
    scratch_types=[
        pltpu.VMEM((NCHUNK, CH), jnp.int32),
        pltpu.VMEM((NCHUNK, CH), jnp.int32),
        pltpu.VMEM((NBUF, CH, 2 * D), jnp.float32),
        pltpu.VMEM((NBUF, CH, 2 * D), jnp.float32),
        pltpu.SemaphoreType.DMA((NBUF,)),
        pltpu.SemaphoreType.DMA((NBUF,)),
    ],
)
def _gather_k(td_hbm, ts_hbm, dst_hbm, src_hbm, g_hbm, idxd, idxs,
              bufd, bufs, sg, sw):
    wid = lax.axis_index("s") * NC + lax.axis_index("c")
    pltpu.sync_copy(dst_hbm.at[wid], idxd)
    pltpu.sync_copy(src_hbm.at[wid], idxs)
    base = wid * EPW

    def gathers(c, slot):
        return (
            pltpu.make_async_copy(td_hbm.at[idxd.at[c]], bufd.at[slot],
                                  sg.at[slot]),
            pltpu.make_async_copy(ts_hbm.at[idxs.at[c]], bufs.at[slot],
                                  sg.at[slot]),
        )

    def write(c, slot):
        rows = pl.ds(base + c * CH, CH)
        return pltpu.make_async_copy(bufd.at[slot], g_hbm.at[rows],
                                     sw.at[slot])

    def add_into(slot):
        def add_row(r, _):
            for j in range(2 * D // 16):
                sl = pl.ds(j * 16, 16)
                bufd[slot, r, sl] = bufd[slot, r, sl] + bufs[slot, r, sl]
            return 0

        lax.fori_loop(0, CH, add_row, 0, unroll=2)

    for i in range(AHEAD):
        ga, gb = gathers(i, i)
        ga.start()
        gb.start()

    def outer(cc, _):
        for i in range(NBUF):
            c = cc * NBUF + i
            ga, gb = gathers(c, i)
            ga.wait()
            gb.wait()
            add_into(i)
            write(c, i).start()
            j = (i + AHEAD) % NBUF
            cn = c + AHEAD

            @pl.when(cn >= NBUF)
            def _drain():
                write(cn - NBUF, j).wait()

            @pl.when(cn < NCMAIN)
            def _prefetch():
                na, nb = gathers(cn, j)
                na.start()
                nb.start()
        return 0

    lax.fori_loop(0, NCMAIN // NBUF, outer, 0)
    for k in range(NBUF - AHEAD):
        c = NCMAIN - (NBUF - AHEAD) + k
        write(c, c % NBUF).wait()
    for t in range(NCHUNK - NCMAIN):
        c = NCMAIN + t
        ga, gb = gathers(c, t)
        ga.start()
        gb.start()
        ga.wait()
        gb.wait()
        add_into(t)
        write(c, t).start()
        write(c, t).wait()


def _zz_body(g_ref, ea_ref, we_ref, zz_ref):
    ew = jnp.dot(ea_ref[...].astype(jnp.bfloat16),
                 we_ref[...].astype(jnp.bfloat16),
                 preferred_element_type=jnp.float32)
    zz_ref[...] = g_ref[...] + ew


def _edge_zz(g, ea, we):
    bm = 4000
    return pl.pallas_call(
        _zz_body,
        grid=(E // bm,),
        in_specs=[
            pl.BlockSpec((bm, 2 * D), lambda i: (i, 0)),
            pl.BlockSpec((bm, DE), lambda i: (i, 0)),
            pl.BlockSpec((DE, 2 * D), lambda i: (0, 0)),
        ],
        out_specs=pl.BlockSpec((bm, 2 * D), lambda i: (i, 0)),
        out_shape=jax.ShapeDtypeStruct((E, 2 * D), jnp.float32),
    )(g, ea, we)


@functools.partial(
    pl.kernel,
    out_type=jax.ShapeDtypeStruct((NC, N, D), jnp.float32),
    mesh=_mesh,
    scratch_types=[
        pltpu.VMEM((NCHS, CHS), jnp.int32),
        pltpu.VMEM((CHS, D), jnp.float32),
        pltpu.VMEM((ZR, D), jnp.float32),
        pltpu.VMEM_SHARED((N, D), jnp.float32),
        pltpu.SemaphoreType.DMA,
    ],
)
def _scatter_k(msg_hbm, dst_hbm, aggp_hbm, idxd, mbuf, zbuf, aggsh, sem):
    cid = lax.axis_index("c")
    sid = lax.axis_index("s")
    wid = sid * NC + cid

    def zrow(i, _):
        for j in range(D // 16):
            zbuf[i, pl.ds(j * 16, 16)] = jnp.zeros((16,), jnp.float32)
        return 0

    lax.fori_loop(0, ZR, zrow, 0)

    @pl.when(sid < WTILES)
    def _init():
        for t in range(RPS // ZR):
            pltpu.sync_copy(zbuf, aggsh.at[pl.ds(sid * RPS + t * ZR, ZR)])

    plsc.subcore_barrier()

    pltpu.sync_copy(dst_hbm.at[wid], idxd)

    def chunk(c, _):
        pltpu.sync_copy(msg_hbm.at[pl.ds(wid * EPW + c * CHS, CHS)], mbuf)
        pltpu.sync_copy(mbuf, aggsh.at[idxd.at[c]], add=True)
        return 0

    lax.fori_loop(0, NCHS, chunk, 0)
    plsc.subcore_barrier()

    @pl.when(sid < WTILES)
    def _writeout():
        pltpu.sync_copy(
            aggsh.at[pl.ds(sid * RPS, RPS)],
            aggp_hbm.at[cid, pl.ds(sid * RPS, RPS)],
        )


def _head1_body(h_ref, w1_ref, b1_ref, z_ref):
    z_ref[...] = (
        jnp.dot(h_ref[...].astype(jnp.bfloat16),
                w1_ref[...].astype(jnp.bfloat16),
                preferred_element_type=jnp.float32) + b1_ref[...]
    )


def _head1(h, w1, b1):
    return pl.pallas_call(
        _head1_body,
        out_shape=jax.ShapeDtypeStruct((N, H), jnp.float32),
    )(h, w1, b1)


def _head2_body(h2_ref, b_ref, w2_ref, b2_ref, out_ref):
    oh = (
        b_ref[...] == lax.broadcasted_iota(jnp.int32, (N, G), 1)
    ).astype(jnp.float32)
    pooled = lax.dot_general(
        oh, h2_ref[...], (((0,), (0,)), ((), ())),
        preferred_element_type=jnp.float32,
        precision=lax.Precision.HIGHEST,
    )
    out = jnp.dot(pooled.astype(jnp.bfloat16), w2_ref[...].astype(jnp.bfloat16),
                  preferred_element_type=jnp.float32)
    out_ref[...] = jnp.maximum(out + b2_ref[...], 0.0)


def _head2(h2, batch2d, w2, b2):
    return pl.pallas_call(
        _head2_body,
        out_shape=jax.ShapeDtypeStruct((G, 1), jnp.float32),
    )(h2, batch2d, w2, b2)


def kernel(x, edge_index, edge_attr, batch, Wf, bf, Ws, bs, W1, b1, W2, b2):
    src = edge_index[0]
    dst = edge_index[1]
    wd = jnp.concatenate([Wf[:D], Ws[:D]], axis=1)
    wsr = jnp.concatenate([Wf[D : 2 * D], Ws[D : 2 * D]], axis=1)
    we = jnp.concatenate([Wf[2 * D :], Ws[2 * D :]], axis=1)
    bcat = jnp.concatenate([bf, bs]).reshape(1, 2 * D)
    td, ts = _tables(x, wd, wsr, bcat)
    dst3 = dst.reshape(NW, NCHUNK, CH)
    src3 = src.reshape(NW, NCHUNK, CH)
    g = _gather_k(td, ts, dst3, src3)
    zz = _edge_zz(g, edge_attr, we)
    msg = jax.nn.sigmoid(zz[:, :D]) * jax.nn.softplus(zz[:, D:])
    aggp = _scatter_k(msg, dst.reshape(NW, NCHS, CHS))
    h = jax.nn.sigmoid(x + aggp[0] + aggp[1])
    z2 = _head1(h, W1, b1.reshape(1, H))
    h2 = jax.nn.sigmoid(z2)
    return _head2(h2, batch.reshape(N, 1), W2, b2.reshape(1, 1))
